# baseline (speedup 1.0000x reference)
import functools

import jax
import jax.numpy as jnp
import numpy as np
from jax import lax
from jax.experimental import pallas as pl
from jax.experimental.pallas import tpu as pltpu
from jax.experimental.pallas import tpu_sc as plsc

N = 10000
E = 320000
H = 128
HW = H // 2
G = 128
BN_INV = float(1.0 / np.sqrt(1.0 + 1e-5))

NC = 2
NS = 16
NW = NC * NS
K = 72
NBUF = 3
CHUNKS = 141
G_ITER = CHUNKS // NBUF
EPB = K * CHUNKS
E_PAD = EPB * NW
ROWS_PT = 640
AGG_ROWS = NS * ROWS_PT

BLK = 200
NBLK = N // BLK
EROWS = E_PAD // 128
EBLK = 256
NEBLK = -(-EROWS // EBLK)

_f32 = jnp.float32


def _pack_rows(z):
    zb = jax.lax.bitcast_convert_type(z, jnp.int32) + 0x8000
    lo = jax.lax.shift_right_logical(zb[:, :HW], 16)
    hi = zb[:, HW:] & jnp.int32(-65536)
    return lo | hi


def _eidx_body(src_ref, attr_ref, gidx_ref):
    code = attr_ref[0] * 4 + attr_ref[1] * 2 + attr_ref[2]
    gidx_ref[...] = code * N + src_ref[...]


_eidx = pl.pallas_call(
    _eidx_body,
    grid=(NEBLK,),
    in_specs=[
        pl.BlockSpec((EBLK, 128), lambda b: (b, 0)),
        pl.BlockSpec((3, EBLK, 128), lambda b: (0, b, 0)),
    ],
    out_specs=pl.BlockSpec((EBLK, 128), lambda b: (b, 0)),
    out_shape=jax.ShapeDtypeStruct((EROWS, 128), jnp.int32),
)


def _enc_body(xf_ref, d_ref, base_ref, t_ref, z_ref, zp_ref):
    z = jnp.dot(xf_ref[...], d_ref[...], preferred_element_type=_f32) + base_ref[...]
    z_ref[...] = z
    for c in range(8):
        zp_ref[c] = _pack_rows(jnp.maximum(z + t_ref[c], 0.0))


_enc = pl.pallas_call(
    _enc_body,
    grid=(NBLK,),
    in_specs=[
        pl.BlockSpec((BLK, 16), lambda b: (b, 0)),
        pl.BlockSpec((16, H), lambda b: (0, 0)),
        pl.BlockSpec((1, H), lambda b: (0, 0)),
        pl.BlockSpec((8, H), lambda b: (0, 0)),
    ],
    out_specs=[
        pl.BlockSpec((BLK, H), lambda b: (b, 0)),
        pl.BlockSpec((8, BLK, HW), lambda b: (0, b, 0)),
    ],
    out_shape=[
        jax.ShapeDtypeStruct((N, H), _f32),
        jax.ShapeDtypeStruct((8, N, HW), jnp.int32),
    ],
)


def _sc_body(gidx_hbm, dst_hbm, zplus_hbm, out_hbm, *scr):
    zbuf = scr[0]
    gb = scr[1:1 + NBUF]
    db = scr[1 + NBUF:1 + 2 * NBUF]
    rb = scr[1 + 2 * NBUF:1 + 3 * NBUF]
    ob = scr[1 + 3 * NBUF:1 + 4 * NBUF]
    agg = scr[1 + 4 * NBUF]
    isg, isd, gsm, ssm = scr[2 + 4 * NBUF:6 + 4 * NBUF]

    cid = lax.axis_index("c")
    sid = lax.axis_index("s")
    wid = cid * NS + sid
    base_e = wid * EPB

    zero16 = jnp.zeros((16,), _f32)

    def _zb(i, c):
        zbuf[i // 8, pl.ds((i % 8) * 16, 16)] = zero16
        return c

    lax.fori_loop(0, 16 * 8, _zb, 0)
    row0 = sid * ROWS_PT

    def _za(r, c):
        pltpu.sync_copy(zbuf, agg.at[pl.ds(row0 + r * 16, 16), :])
        return c

    lax.fori_loop(0, ROWS_PT // 16, _za, 0)
    plsc.subcore_barrier()

    for b in range(NBUF):
        off = base_e + b * K
        pltpu.async_copy(gidx_hbm.at[pl.ds(off, K)], gb[b], isg.at[b])
        pltpu.async_copy(dst_hbm.at[pl.ds(off, K)], db[b], isd.at[b])

    def _group(g, c):
        e0 = base_e + g * (NBUF * K)
        gds = []
        for b in range(NBUF):
            pltpu.make_async_copy(
                gidx_hbm.at[pl.ds(e0 + b * K, K)], gb[b], isg.at[b]).wait()
            pltpu.make_async_copy(
                dst_hbm.at[pl.ds(e0 + b * K, K)], db[b], isd.at[b]).wait()
            gds.append(pltpu.async_copy(zplus_hbm.at[gb[b]], rb[b], gsm.at[b]))
        mask_hi = jnp.full((16,), -65536, jnp.int32)
        sds = []
        for b in range(NBUF):
            gds[b].wait()

            def _edge4(j4, cc, _b=b):
                for u in range(4):
                    j = j4 * 4 + u
                    for q in range(4):
                        sl = pl.ds(q * 16, 16)
                        sh = pl.ds(HW + q * 16, 16)
                        w = rb[_b][j, sl]
                        ob[_b][j, sl] = plsc.bitcast(w << 16, _f32)
                        ob[_b][j, sh] = plsc.bitcast(w & mask_hi, _f32)
                return cc

            lax.fori_loop(0, K // 4, _edge4, 0)
            sds.append(pltpu.async_copy(ob[b], agg.at[db[b]], ssm.at[b], add=True))
        for b in range(NBUF):
            sds[b].wait()

        @pl.when(g < G_ITER - 1)
        def _():
            e1 = e0 + NBUF * K
            for b in range(NBUF):
                pltpu.async_copy(gidx_hbm.at[pl.ds(e1 + b * K, K)], gb[b], isg.at[b])
                pltpu.async_copy(dst_hbm.at[pl.ds(e1 + b * K, K)], db[b], isd.at[b])

        return c

    lax.fori_loop(0, G_ITER, _group, 0)
    plsc.subcore_barrier()

    pltpu.sync_copy(agg.at[pl.ds(row0, ROWS_PT), :],
                    out_hbm.at[cid, pl.ds(row0, ROWS_PT), :])


@functools.lru_cache(maxsize=1)
def _get_sc_agg():
    return functools.partial(
        pl.kernel,
        out_type=jax.ShapeDtypeStruct((NC, AGG_ROWS, H), _f32),
        compiler_params=pltpu.CompilerParams(use_tc_tiling_on_sc=False,
                                             needs_layout_passes=False),
        mesh=plsc.VectorSubcoreMesh(core_axis_name="c", subcore_axis_name="s",
                                    num_cores=NC, num_subcores=NS),
        scratch_types=(
            [pltpu.VMEM((16, H), _f32)]
            + [pltpu.VMEM((K,), jnp.int32) for _ in range(NBUF)]
            + [pltpu.VMEM((K,), jnp.int32) for _ in range(NBUF)]
            + [pltpu.VMEM((K, HW), jnp.int32) for _ in range(NBUF)]
            + [pltpu.VMEM((K, H), _f32) for _ in range(NBUF)]
            + [pltpu.VMEM_SHARED((AGG_ROWS, H), _f32)]
            + [pltpu.SemaphoreType.DMA((NBUF,)) for _ in range(4)]
        ),
    )(_sc_body)


def _mlp_body(last, z_ref, agg_ref, w1_ref, b1_ref, w2_ref, b2_ref, t_ref,
              batch_ref, zout_ref, pool_ref, zp_ref=None):
    b = pl.program_id(0)
    h = z_ref[...] + agg_ref[0] + agg_ref[1]
    h1 = jnp.maximum(
        jnp.dot(h, w1_ref[...], preferred_element_type=_f32) + b1_ref[...], 0.0)
    z2 = jnp.dot(h1, w2_ref[...], preferred_element_type=_f32) + b2_ref[...]
    if not last:
        z2 = jnp.maximum(z2, 0.0)
    zout_ref[...] = z2
    seg = batch_ref[0, 0, :]
    oh_t = (lax.broadcasted_iota(jnp.int32, (G, BLK), 0)
            == seg[None, :]).astype(_f32)
    contrib = jnp.dot(oh_t, z2, preferred_element_type=_f32)

    @pl.when(b == 0)
    def _():
        pool_ref[...] = jnp.zeros_like(pool_ref)

    pool_ref[...] += contrib
    if zp_ref is not None:
        for c in range(8):
            zp_ref[c] = _pack_rows(jnp.maximum(z2 + t_ref[c], 0.0))


def _make_mlp(last):
    out_specs = [
        pl.BlockSpec((BLK, H), lambda b: (b, 0)),
        pl.BlockSpec((G, H), lambda b: (0, 0)),
    ]
    out_shape = [
        jax.ShapeDtypeStruct((N, H), _f32),
        jax.ShapeDtypeStruct((G, H), _f32),
    ]
    if not last:
        out_specs.append(pl.BlockSpec((8, BLK, HW), lambda b: (0, b, 0)))
        out_shape.append(jax.ShapeDtypeStruct((8, N, HW), jnp.int32))
    return pl.pallas_call(
        functools.partial(_mlp_body, last),
        grid=(NBLK,),
        in_specs=[
            pl.BlockSpec((BLK, H), lambda b: (b, 0)),
            pl.BlockSpec((NC, BLK, H), lambda b: (0, b, 0)),
            pl.BlockSpec((H, 2 * H), lambda b: (0, 0)),
            pl.BlockSpec((1, 2 * H), lambda b: (0, 0)),
            pl.BlockSpec((2 * H, H), lambda b: (0, 0)),
            pl.BlockSpec((1, H), lambda b: (0, 0)),
            pl.BlockSpec((8, H), lambda b: (0, 0)),
            pl.BlockSpec((1, 1, BLK), lambda b: (b, 0, 0)),
        ],
        out_specs=out_specs,
        out_shape=out_shape,
    )


_mlp_mid = _make_mlp(False)
_mlp_last = _make_mlp(True)


def kernel(x, edge_index, edge_attr, batch, params):
    src = edge_index[0]
    dst = edge_index[1]
    srcp = jnp.pad(src, (0, E_PAD - E))
    dstp = jnp.pad(dst, (0, E_PAD - E), constant_values=N)
    attrp = jnp.pad(edge_attr.T, ((0, 0), (0, E_PAD - E)))
    gidx = _eidx(srcp.reshape(EROWS, 128),
                 attrp.reshape(3, EROWS, 128)).reshape(E_PAD)

    xf = jnp.pad(x.astype(_f32), ((0, 0), (0, 7)))
    d_mat = jnp.pad(
        jnp.stack([params['atom_emb'][i][1] - params['atom_emb'][i][0]
                   for i in range(9)]), ((0, 7), (0, 0)))
    base = sum(params['atom_emb'][i][0] for i in range(9))[None, :]
    b0, b1, b2 = params['bond_emb']
    i0 = jnp.array([0, 0, 0, 0, 1, 1, 1, 1], jnp.int32)
    i1 = jnp.array([0, 0, 1, 1, 0, 0, 1, 1], jnp.int32)
    i2 = jnp.array([0, 1, 0, 1, 0, 1, 0, 1], jnp.int32)
    t_tab = b0[i0] + b1[i1] + b2[i2]

    zcur, zpcur = _enc(xf, d_mat, base, t_tab)
    batch3 = batch.reshape(NBLK, 1, BLK)

    zs, gs = [], []
    for li, p in enumerate(params['layers']):
        s1 = BN_INV * p['g1']
        w1f = p['W1'] * s1[None, :]
        b1f = (p['b1'] * s1 + p['be1'])[None, :]
        s2 = BN_INV * p['g']
        w2f = p['W2'] * s2[None, :]
        b2f = (p['b2'] * s2 + p['be'])[None, :]
        agg = _get_sc_agg()(gidx, dstp, zpcur.reshape(8 * N, HW))
        if li < 2:
            zcur, pool, zpcur = _mlp_mid(zcur, agg, w1f, b1f, w2f, b2f,
                                         t_tab, batch3)
        else:
            zcur, pool = _mlp_last(zcur, agg, w1f, b1f, w2f, b2f,
                                   t_tab, batch3)
        zs.append(zcur)
        gs.append(pool)
    return jnp.concatenate(zs, axis=1), jnp.concatenate(gs, axis=1)


# speedup vs baseline: 1.0288x; 1.0288x over previous
import functools

import jax
import jax.numpy as jnp
import numpy as np
from jax import lax
from jax.experimental import pallas as pl
from jax.experimental.pallas import tpu as pltpu
from jax.experimental.pallas import tpu_sc as plsc

N = 10000
E = 320000
H = 128
HW = H // 2
G = 128
BN_INV = float(1.0 / np.sqrt(1.0 + 1e-5))

NC = 2
NS = 16
NW = NC * NS
K = 112
NBUF = 2
CHUNKS = 90
G_ITER = CHUNKS // NBUF
EPB = K * CHUNKS
E_PAD = EPB * NW
ROWS_PT = 640
AGG_ROWS = NS * ROWS_PT

BLK = 200
NBLK = N // BLK
EROWS = E_PAD // 128
EBLK = 256
NEBLK = -(-EROWS // EBLK)

_f32 = jnp.float32


def _pack_rows(z):
    zb = jax.lax.bitcast_convert_type(z, jnp.int32) + 0x8000
    lo = jax.lax.shift_right_logical(zb[:, :HW], 16)
    hi = zb[:, HW:] & jnp.int32(-65536)
    return lo | hi


def _eidx_body(src_ref, attr_ref, gidx_ref):
    code = attr_ref[0] * 4 + attr_ref[1] * 2 + attr_ref[2]
    gidx_ref[...] = code * N + src_ref[...]


_eidx = pl.pallas_call(
    _eidx_body,
    grid=(NEBLK,),
    in_specs=[
        pl.BlockSpec((EBLK, 128), lambda b: (b, 0)),
        pl.BlockSpec((3, EBLK, 128), lambda b: (0, b, 0)),
    ],
    out_specs=pl.BlockSpec((EBLK, 128), lambda b: (b, 0)),
    out_shape=jax.ShapeDtypeStruct((EROWS, 128), jnp.int32),
)


def _enc_body(xf_ref, d_ref, base_ref, t_ref, z_ref, zp_ref):
    z = jnp.dot(xf_ref[...], d_ref[...], preferred_element_type=_f32) + base_ref[...]
    z_ref[...] = z
    for c in range(8):
        zp_ref[c] = _pack_rows(jnp.maximum(z + t_ref[c], 0.0))


_enc = pl.pallas_call(
    _enc_body,
    grid=(NBLK,),
    in_specs=[
        pl.BlockSpec((BLK, 16), lambda b: (b, 0)),
        pl.BlockSpec((16, H), lambda b: (0, 0)),
        pl.BlockSpec((1, H), lambda b: (0, 0)),
        pl.BlockSpec((8, H), lambda b: (0, 0)),
    ],
    out_specs=[
        pl.BlockSpec((BLK, H), lambda b: (b, 0)),
        pl.BlockSpec((8, BLK, HW), lambda b: (0, b, 0)),
    ],
    out_shape=[
        jax.ShapeDtypeStruct((N, H), _f32),
        jax.ShapeDtypeStruct((8, N, HW), jnp.int32),
    ],
)


def _sc_body(gidx_hbm, dst_hbm, zplus_hbm, out_hbm, *scr):
    zbuf = scr[0]
    gb = scr[1:1 + NBUF]
    db = scr[1 + NBUF:1 + 2 * NBUF]
    rb = scr[1 + 2 * NBUF:1 + 3 * NBUF]
    ob = scr[1 + 3 * NBUF:1 + 4 * NBUF]
    agg = scr[1 + 4 * NBUF]
    isg, isd, gsm, ssm = scr[2 + 4 * NBUF:6 + 4 * NBUF]

    cid = lax.axis_index("c")
    sid = lax.axis_index("s")
    wid = cid * NS + sid
    base_e = wid * EPB

    zero16 = jnp.zeros((16,), _f32)

    def _zb(i, c):
        zbuf[i // 8, pl.ds((i % 8) * 16, 16)] = zero16
        return c

    lax.fori_loop(0, 16 * 8, _zb, 0)
    row0 = sid * ROWS_PT

    def _za(r, c):
        pltpu.sync_copy(zbuf, agg.at[pl.ds(row0 + r * 16, 16), :])
        return c

    lax.fori_loop(0, ROWS_PT // 16, _za, 0)
    plsc.subcore_barrier()

    for b in range(NBUF):
        off = base_e + b * K
        pltpu.async_copy(gidx_hbm.at[pl.ds(off, K)], gb[b], isg.at[b])
        pltpu.async_copy(dst_hbm.at[pl.ds(off, K)], db[b], isd.at[b])

    def _group(g, c):
        e0 = base_e + g * (NBUF * K)
        gds = []
        for b in range(NBUF):
            pltpu.make_async_copy(
                gidx_hbm.at[pl.ds(e0 + b * K, K)], gb[b], isg.at[b]).wait()
            pltpu.make_async_copy(
                dst_hbm.at[pl.ds(e0 + b * K, K)], db[b], isd.at[b]).wait()
            gds.append(pltpu.async_copy(zplus_hbm.at[gb[b]], rb[b], gsm.at[b]))
        mask_hi = jnp.full((16,), -65536, jnp.int32)
        sds = []
        for b in range(NBUF):
            gds[b].wait()

            def _edge4(j4, cc, _b=b):
                for u in range(4):
                    j = j4 * 4 + u
                    for q in range(4):
                        sl = pl.ds(q * 16, 16)
                        sh = pl.ds(HW + q * 16, 16)
                        w = rb[_b][j, sl]
                        ob[_b][j, sl] = plsc.bitcast(w << 16, _f32)
                        ob[_b][j, sh] = plsc.bitcast(w & mask_hi, _f32)
                return cc

            lax.fori_loop(0, K // 4, _edge4, 0)
            sds.append(pltpu.async_copy(ob[b], agg.at[db[b]], ssm.at[b], add=True))
        for b in range(NBUF):
            sds[b].wait()

        @pl.when(g < G_ITER - 1)
        def _():
            e1 = e0 + NBUF * K
            for b in range(NBUF):
                pltpu.async_copy(gidx_hbm.at[pl.ds(e1 + b * K, K)], gb[b], isg.at[b])
                pltpu.async_copy(dst_hbm.at[pl.ds(e1 + b * K, K)], db[b], isd.at[b])

        return c

    lax.fori_loop(0, G_ITER, _group, 0)
    plsc.subcore_barrier()

    pltpu.sync_copy(agg.at[pl.ds(row0, ROWS_PT), :],
                    out_hbm.at[cid, pl.ds(row0, ROWS_PT), :])


@functools.lru_cache(maxsize=1)
def _get_sc_agg():
    return functools.partial(
        pl.kernel,
        out_type=jax.ShapeDtypeStruct((NC, AGG_ROWS, H), _f32),
        compiler_params=pltpu.CompilerParams(use_tc_tiling_on_sc=False,
                                             needs_layout_passes=False),
        mesh=plsc.VectorSubcoreMesh(core_axis_name="c", subcore_axis_name="s",
                                    num_cores=NC, num_subcores=NS),
        scratch_types=(
            [pltpu.VMEM((16, H), _f32)]
            + [pltpu.VMEM((K,), jnp.int32) for _ in range(NBUF)]
            + [pltpu.VMEM((K,), jnp.int32) for _ in range(NBUF)]
            + [pltpu.VMEM((K, HW), jnp.int32) for _ in range(NBUF)]
            + [pltpu.VMEM((K, H), _f32) for _ in range(NBUF)]
            + [pltpu.VMEM_SHARED((AGG_ROWS, H), _f32)]
            + [pltpu.SemaphoreType.DMA((NBUF,)) for _ in range(4)]
        ),
    )(_sc_body)


def _mlp_body(last, z_ref, agg_ref, w1_ref, b1_ref, w2_ref, b2_ref, t_ref,
              batch_ref, zout_ref, pool_ref, zp_ref=None):
    b = pl.program_id(0)
    h = z_ref[...] + agg_ref[0] + agg_ref[1]
    h1 = jnp.maximum(
        jnp.dot(h, w1_ref[...], preferred_element_type=_f32) + b1_ref[...], 0.0)
    z2 = jnp.dot(h1, w2_ref[...], preferred_element_type=_f32) + b2_ref[...]
    if not last:
        z2 = jnp.maximum(z2, 0.0)
    zout_ref[...] = z2
    seg = batch_ref[0, 0, :]
    oh_t = (lax.broadcasted_iota(jnp.int32, (G, BLK), 0)
            == seg[None, :]).astype(_f32)
    contrib = jnp.dot(oh_t, z2, preferred_element_type=_f32)

    @pl.when(b == 0)
    def _():
        pool_ref[...] = jnp.zeros_like(pool_ref)

    pool_ref[...] += contrib
    if zp_ref is not None:
        for c in range(8):
            zp_ref[c] = _pack_rows(jnp.maximum(z2 + t_ref[c], 0.0))


def _make_mlp(last):
    out_specs = [
        pl.BlockSpec((BLK, H), lambda b: (b, 0)),
        pl.BlockSpec((G, H), lambda b: (0, 0)),
    ]
    out_shape = [
        jax.ShapeDtypeStruct((N, H), _f32),
        jax.ShapeDtypeStruct((G, H), _f32),
    ]
    if not last:
        out_specs.append(pl.BlockSpec((8, BLK, HW), lambda b: (0, b, 0)))
        out_shape.append(jax.ShapeDtypeStruct((8, N, HW), jnp.int32))
    return pl.pallas_call(
        functools.partial(_mlp_body, last),
        grid=(NBLK,),
        in_specs=[
            pl.BlockSpec((BLK, H), lambda b: (b, 0)),
            pl.BlockSpec((NC, BLK, H), lambda b: (0, b, 0)),
            pl.BlockSpec((H, 2 * H), lambda b: (0, 0)),
            pl.BlockSpec((1, 2 * H), lambda b: (0, 0)),
            pl.BlockSpec((2 * H, H), lambda b: (0, 0)),
            pl.BlockSpec((1, H), lambda b: (0, 0)),
            pl.BlockSpec((8, H), lambda b: (0, 0)),
            pl.BlockSpec((1, 1, BLK), lambda b: (b, 0, 0)),
        ],
        out_specs=out_specs,
        out_shape=out_shape,
    )


_mlp_mid = _make_mlp(False)
_mlp_last = _make_mlp(True)


def kernel(x, edge_index, edge_attr, batch, params):
    src = edge_index[0]
    dst = edge_index[1]
    srcp = jnp.pad(src, (0, E_PAD - E))
    dstp = jnp.pad(dst, (0, E_PAD - E), constant_values=N)
    attrp = jnp.pad(edge_attr.T, ((0, 0), (0, E_PAD - E)))
    gidx = _eidx(srcp.reshape(EROWS, 128),
                 attrp.reshape(3, EROWS, 128)).reshape(E_PAD)

    xf = jnp.pad(x.astype(_f32), ((0, 0), (0, 7)))
    d_mat = jnp.pad(
        jnp.stack([params['atom_emb'][i][1] - params['atom_emb'][i][0]
                   for i in range(9)]), ((0, 7), (0, 0)))
    base = sum(params['atom_emb'][i][0] for i in range(9))[None, :]
    b0, b1, b2 = params['bond_emb']
    i0 = jnp.array([0, 0, 0, 0, 1, 1, 1, 1], jnp.int32)
    i1 = jnp.array([0, 0, 1, 1, 0, 0, 1, 1], jnp.int32)
    i2 = jnp.array([0, 1, 0, 1, 0, 1, 0, 1], jnp.int32)
    t_tab = b0[i0] + b1[i1] + b2[i2]

    zcur, zpcur = _enc(xf, d_mat, base, t_tab)
    batch3 = batch.reshape(NBLK, 1, BLK)

    zs, gs = [], []
    for li, p in enumerate(params['layers']):
        s1 = BN_INV * p['g1']
        w1f = p['W1'] * s1[None, :]
        b1f = (p['b1'] * s1 + p['be1'])[None, :]
        s2 = BN_INV * p['g']
        w2f = p['W2'] * s2[None, :]
        b2f = (p['b2'] * s2 + p['be'])[None, :]
        agg = _get_sc_agg()(gidx, dstp, zpcur.reshape(8 * N, HW))
        if li < 2:
            zcur, pool, zpcur = _mlp_mid(zcur, agg, w1f, b1f, w2f, b2f,
                                         t_tab, batch3)
        else:
            zcur, pool = _mlp_last(zcur, agg, w1f, b1f, w2f, b2f,
                                   t_tab, batch3)
        zs.append(zcur)
        gs.append(pool)
    return jnp.concatenate(zs, axis=1), jnp.concatenate(gs, axis=1)


# speedup vs baseline: 1.1750x; 1.1421x over previous
import functools

import jax
import jax.numpy as jnp
import numpy as np
from jax import lax
from jax.experimental import pallas as pl
from jax.experimental.pallas import tpu as pltpu
from jax.experimental.pallas import tpu_sc as plsc

N = 10000
E = 320000
H = 128
HW = H // 2
G = 128
BN_INV = float(1.0 / np.sqrt(1.0 + 1e-5))

NC = 2
NS = 16
NW = NC * NS
K = 112
NBUF = 2
CHUNKS = 90
G_ITER = CHUNKS // NBUF
EPB = K * CHUNKS
E_PAD = EPB * NW
ROWS_PT = 640
AGG_ROWS = NS * ROWS_PT

BLK = 200
NBLK = N // BLK
EROWS = E_PAD // 128
EBLK = 256
NEBLK = -(-EROWS // EBLK)

_f32 = jnp.float32


def _pack_rows(z):
    zb = jax.lax.bitcast_convert_type(z, jnp.int32) + 0x8000
    lo = jax.lax.shift_right_logical(zb[:, :HW], 16)
    hi = zb[:, HW:] & jnp.int32(-65536)
    return lo | hi


def _eidx_body(src_ref, attr_ref, gidx_ref):
    code = attr_ref[0] * 4 + attr_ref[1] * 2 + attr_ref[2]
    gidx_ref[...] = code * N + src_ref[...]


_eidx = pl.pallas_call(
    _eidx_body,
    grid=(NEBLK,),
    in_specs=[
        pl.BlockSpec((EBLK, 128), lambda b: (b, 0)),
        pl.BlockSpec((3, EBLK, 128), lambda b: (0, b, 0)),
    ],
    out_specs=pl.BlockSpec((EBLK, 128), lambda b: (b, 0)),
    out_shape=jax.ShapeDtypeStruct((EROWS, 128), jnp.int32),
)


def _enc_body(xf_ref, d_ref, base_ref, t_ref, z_ref, zp_ref):
    z = jnp.dot(xf_ref[...], d_ref[...], preferred_element_type=_f32) + base_ref[...]
    z_ref[...] = z
    for c in range(8):
        zp_ref[c] = _pack_rows(jnp.maximum(z + t_ref[c], 0.0))


_enc = pl.pallas_call(
    _enc_body,
    grid=(NBLK,),
    in_specs=[
        pl.BlockSpec((BLK, 16), lambda b: (b, 0)),
        pl.BlockSpec((16, H), lambda b: (0, 0)),
        pl.BlockSpec((1, H), lambda b: (0, 0)),
        pl.BlockSpec((8, H), lambda b: (0, 0)),
    ],
    out_specs=[
        pl.BlockSpec((BLK, H), lambda b: (b, 0)),
        pl.BlockSpec((8, BLK, HW), lambda b: (0, b, 0)),
    ],
    out_shape=[
        jax.ShapeDtypeStruct((N, H), _f32),
        jax.ShapeDtypeStruct((8, N, HW), jnp.int32),
    ],
)


def _sc_body(gidx_hbm, dst_hbm, zplus_hbm, out_hbm, *scr):
    zbuf = scr[0]
    gb = scr[1:1 + NBUF]
    db = scr[1 + NBUF:1 + 2 * NBUF]
    rb = scr[1 + 2 * NBUF:1 + 3 * NBUF]
    ob = scr[1 + 3 * NBUF:1 + 4 * NBUF]
    agg = scr[1 + 4 * NBUF]
    isg, isd, gsm, ssm = scr[2 + 4 * NBUF:6 + 4 * NBUF]

    cid = lax.axis_index("c")
    sid = lax.axis_index("s")
    wid = cid * NS + sid
    base_e = wid * EPB

    zero16 = jnp.zeros((16,), _f32)

    def _zb(i, c):
        zbuf[i // 8, pl.ds((i % 8) * 16, 16)] = zero16
        return c

    lax.fori_loop(0, 16 * 8, _zb, 0)
    row0 = sid * ROWS_PT

    def _za(r, c):
        pltpu.sync_copy(zbuf, agg.at[pl.ds(row0 + r * 16, 16), :])
        return c

    lax.fori_loop(0, ROWS_PT // 16, _za, 0)
    plsc.subcore_barrier()

    for b in range(NBUF):
        off = base_e + b * K
        pltpu.async_copy(gidx_hbm.at[pl.ds(off, K)], gb[b], isg.at[b])
        pltpu.async_copy(dst_hbm.at[pl.ds(off, K)], db[b], isd.at[b])

    def _group(g, c):
        e0 = base_e + g * (NBUF * K)
        gds = []
        for b in range(NBUF):
            pltpu.make_async_copy(
                gidx_hbm.at[pl.ds(e0 + b * K, K)], gb[b], isg.at[b]).wait()
            pltpu.make_async_copy(
                dst_hbm.at[pl.ds(e0 + b * K, K)], db[b], isd.at[b]).wait()
            gds.append(pltpu.async_copy(zplus_hbm.at[gb[b]], rb[b], gsm.at[b]))
        mask_hi = jnp.full((16,), -65536, jnp.int32)
        sds = []
        for b in range(NBUF):
            gds[b].wait()

            def _edge4(j4, cc, _b=b):
                for u in range(4):
                    j = j4 * 4 + u
                    for q in range(4):
                        sl = pl.ds(q * 16, 16)
                        sh = pl.ds(HW + q * 16, 16)
                        w = rb[_b][j, sl]
                        ob[_b][j, sl] = plsc.bitcast(w << 16, _f32)
                        ob[_b][j, sh] = plsc.bitcast(w & mask_hi, _f32)
                return cc

            lax.fori_loop(0, K // 4, _edge4, 0)
            sds.append(pltpu.async_copy(ob[b], agg.at[db[b]], ssm.at[b], add=True))
        for b in range(NBUF):
            sds[b].wait()

        @pl.when(g < G_ITER - 1)
        def _():
            e1 = e0 + NBUF * K
            for b in range(NBUF):
                pltpu.async_copy(gidx_hbm.at[pl.ds(e1 + b * K, K)], gb[b], isg.at[b])
                pltpu.async_copy(dst_hbm.at[pl.ds(e1 + b * K, K)], db[b], isd.at[b])

        return c

    lax.fori_loop(0, G_ITER, _group, 0)
    plsc.subcore_barrier()

    pltpu.sync_copy(agg.at[pl.ds(row0, ROWS_PT), :],
                    out_hbm.at[cid, pl.ds(row0, ROWS_PT), :])


@functools.lru_cache(maxsize=1)
def _get_sc_agg():
    return functools.partial(
        pl.kernel,
        out_type=jax.ShapeDtypeStruct((NC, AGG_ROWS, H), _f32),
        compiler_params=pltpu.CompilerParams(use_tc_tiling_on_sc=False,
                                             needs_layout_passes=False),
        mesh=plsc.VectorSubcoreMesh(core_axis_name="c", subcore_axis_name="s",
                                    num_cores=NC, num_subcores=NS),
        scratch_types=(
            [pltpu.VMEM((16, H), _f32)]
            + [pltpu.VMEM((K,), jnp.int32) for _ in range(NBUF)]
            + [pltpu.VMEM((K,), jnp.int32) for _ in range(NBUF)]
            + [pltpu.VMEM((K, HW), jnp.int32) for _ in range(NBUF)]
            + [pltpu.VMEM((K, H), _f32) for _ in range(NBUF)]
            + [pltpu.VMEM_SHARED((AGG_ROWS, H), _f32)]
            + [pltpu.SemaphoreType.DMA((NBUF,)) for _ in range(4)]
        ),
    )(_sc_body)


def _mlp_body(last, z_ref, agg_ref, w1_ref, b1_ref, w2_ref, b2_ref, t_ref,
              batch_ref, zout_ref, pool_ref, zp_ref=None):
    b = pl.program_id(0)
    h = z_ref[...] + agg_ref[0] + agg_ref[1]
    h1 = jnp.maximum(
        jnp.dot(h, w1_ref[...], preferred_element_type=_f32) + b1_ref[...], 0.0)
    z2 = jnp.dot(h1, w2_ref[...], preferred_element_type=_f32) + b2_ref[...]
    if not last:
        z2 = jnp.maximum(z2, 0.0)
    zout_ref[...] = z2
    seg = batch_ref[0, 0, :]
    oh_t = (lax.broadcasted_iota(jnp.int32, (G, BLK), 0)
            == seg[None, :]).astype(_f32)
    contrib = jnp.dot(oh_t, z2, preferred_element_type=_f32)

    @pl.when(b == 0)
    def _():
        pool_ref[...] = jnp.zeros_like(pool_ref)

    pool_ref[...] += contrib
    if zp_ref is not None:
        for c in range(8):
            zp_ref[c] = _pack_rows(jnp.maximum(z2 + t_ref[c], 0.0))


def _make_mlp(last):
    out_specs = [
        pl.BlockSpec((BLK, H), lambda b: (b, 0)),
        pl.BlockSpec((G, H), lambda b: (0, 0)),
    ]
    out_shape = [
        jax.ShapeDtypeStruct((N, H), _f32),
        jax.ShapeDtypeStruct((G, H), _f32),
    ]
    if not last:
        out_specs.append(pl.BlockSpec((8, BLK, HW), lambda b: (0, b, 0)))
        out_shape.append(jax.ShapeDtypeStruct((8, N, HW), jnp.int32))
    return pl.pallas_call(
        functools.partial(_mlp_body, last),
        grid=(NBLK,),
        in_specs=[
            pl.BlockSpec((BLK, H), lambda b: (b, 0)),
            pl.BlockSpec((NC, BLK, H), lambda b: (0, b, 0)),
            pl.BlockSpec((H, 2 * H), lambda b: (0, 0)),
            pl.BlockSpec((1, 2 * H), lambda b: (0, 0)),
            pl.BlockSpec((2 * H, H), lambda b: (0, 0)),
            pl.BlockSpec((1, H), lambda b: (0, 0)),
            pl.BlockSpec((8, H), lambda b: (0, 0)),
            pl.BlockSpec((1, 1, BLK), lambda b: (b, 0, 0)),
        ],
        out_specs=out_specs,
        out_shape=out_shape,
    )


_mlp_mid = _make_mlp(False)
_mlp_last = _make_mlp(True)


def kernel(x, edge_index, edge_attr, batch, params):
    src = edge_index[0]
    dst = edge_index[1]
    srcp = jnp.concatenate(
        [src, jnp.arange(E_PAD - E, dtype=jnp.int32) % N])
    dstp = jnp.pad(dst, (0, E_PAD - E), constant_values=N)
    attrp = jnp.pad(edge_attr.T, ((0, 0), (0, E_PAD - E)))
    gidx = _eidx(srcp.reshape(EROWS, 128),
                 attrp.reshape(3, EROWS, 128)).reshape(E_PAD)

    xf = jnp.pad(x.astype(_f32), ((0, 0), (0, 7)))
    d_mat = jnp.pad(
        jnp.stack([params['atom_emb'][i][1] - params['atom_emb'][i][0]
                   for i in range(9)]), ((0, 7), (0, 0)))
    base = sum(params['atom_emb'][i][0] for i in range(9))[None, :]
    b0, b1, b2 = params['bond_emb']
    i0 = jnp.array([0, 0, 0, 0, 1, 1, 1, 1], jnp.int32)
    i1 = jnp.array([0, 0, 1, 1, 0, 0, 1, 1], jnp.int32)
    i2 = jnp.array([0, 1, 0, 1, 0, 1, 0, 1], jnp.int32)
    t_tab = b0[i0] + b1[i1] + b2[i2]

    zcur, zpcur = _enc(xf, d_mat, base, t_tab)
    batch3 = batch.reshape(NBLK, 1, BLK)

    zs, gs = [], []
    for li, p in enumerate(params['layers']):
        s1 = BN_INV * p['g1']
        w1f = p['W1'] * s1[None, :]
        b1f = (p['b1'] * s1 + p['be1'])[None, :]
        s2 = BN_INV * p['g']
        w2f = p['W2'] * s2[None, :]
        b2f = (p['b2'] * s2 + p['be'])[None, :]
        agg = _get_sc_agg()(gidx, dstp, zpcur.reshape(8 * N, HW))
        if li < 2:
            zcur, pool, zpcur = _mlp_mid(zcur, agg, w1f, b1f, w2f, b2f,
                                         t_tab, batch3)
        else:
            zcur, pool = _mlp_last(zcur, agg, w1f, b1f, w2f, b2f,
                                   t_tab, batch3)
        zs.append(zcur)
        gs.append(pool)
    return jnp.concatenate(zs, axis=1), jnp.concatenate(gs, axis=1)


# speedup vs baseline: 1.1756x; 1.0005x over previous
import functools

import jax
import jax.numpy as jnp
import numpy as np
from jax import lax
from jax.experimental import pallas as pl
from jax.experimental.pallas import tpu as pltpu
from jax.experimental.pallas import tpu_sc as plsc

N = 10000
E = 320000
H = 128
HW = H // 2
G = 128
BN_INV = float(1.0 / np.sqrt(1.0 + 1e-5))

NC = 2
NS = 16
NW = NC * NS
K = 112
NBUF = 2
CHUNKS = 90
G_ITER = CHUNKS // NBUF
EPB = K * CHUNKS
E_PAD = EPB * NW
ROWS_PT = 640
AGG_ROWS = NS * ROWS_PT

BLK = 200
NBLK = N // BLK
EROWS = E_PAD // 128
EBLK = 256
NEBLK = -(-EROWS // EBLK)

_f32 = jnp.float32


def _pack_rows(z):
    zb = jax.lax.bitcast_convert_type(z, jnp.int32) + 0x8000
    lo = jax.lax.shift_right_logical(zb[:, :HW], 16)
    hi = zb[:, HW:] & jnp.int32(-65536)
    return lo | hi


def _eidx_body(src_ref, attr_ref, gidx_ref):
    code = attr_ref[0] * 4 + attr_ref[1] * 2 + attr_ref[2]
    gidx_ref[...] = code * N + src_ref[...]


_eidx = pl.pallas_call(
    _eidx_body,
    grid=(NEBLK,),
    in_specs=[
        pl.BlockSpec((EBLK, 128), lambda b: (b, 0)),
        pl.BlockSpec((3, EBLK, 128), lambda b: (0, b, 0)),
    ],
    out_specs=pl.BlockSpec((EBLK, 128), lambda b: (b, 0)),
    out_shape=jax.ShapeDtypeStruct((EROWS, 128), jnp.int32),
)


def _enc_body(xf_ref, d_ref, base_ref, t_ref, z_ref, zp_ref):
    z = jnp.dot(xf_ref[...], d_ref[...], preferred_element_type=_f32) + base_ref[...]
    z_ref[...] = z
    for c in range(8):
        zp_ref[c] = _pack_rows(jnp.maximum(z + t_ref[c], 0.0))


_enc = pl.pallas_call(
    _enc_body,
    grid=(NBLK,),
    in_specs=[
        pl.BlockSpec((BLK, 16), lambda b: (b, 0)),
        pl.BlockSpec((16, H), lambda b: (0, 0)),
        pl.BlockSpec((1, H), lambda b: (0, 0)),
        pl.BlockSpec((8, H), lambda b: (0, 0)),
    ],
    out_specs=[
        pl.BlockSpec((BLK, H), lambda b: (b, 0)),
        pl.BlockSpec((8, BLK, HW), lambda b: (0, b, 0)),
    ],
    out_shape=[
        jax.ShapeDtypeStruct((N, H), _f32),
        jax.ShapeDtypeStruct((8, N, HW), jnp.int32),
    ],
)


def _sc_body(gidx_hbm, dst_hbm, zplus_hbm, out_hbm, *scr):
    zbuf = scr[0]
    gb = scr[1:1 + NBUF]
    db = scr[1 + NBUF:1 + 2 * NBUF]
    rb = scr[1 + 2 * NBUF:1 + 3 * NBUF]
    ob = scr[1 + 3 * NBUF:1 + 4 * NBUF]
    agg = scr[1 + 4 * NBUF]
    isg, isd, gsm, ssm = scr[2 + 4 * NBUF:6 + 4 * NBUF]

    cid = lax.axis_index("c")
    sid = lax.axis_index("s")
    wid = cid * NS + sid
    base_e = wid * EPB

    zero16 = jnp.zeros((16,), _f32)

    def _zb(i, c):
        zbuf[i // 8, pl.ds((i % 8) * 16, 16)] = zero16
        return c

    lax.fori_loop(0, 16 * 8, _zb, 0)
    row0 = sid * ROWS_PT

    def _za(r, c):
        pltpu.sync_copy(zbuf, agg.at[pl.ds(row0 + r * 16, 16), :])
        return c

    lax.fori_loop(0, ROWS_PT // 16, _za, 0)
    plsc.subcore_barrier()

    for b in range(NBUF):
        off = base_e + b * K
        pltpu.async_copy(gidx_hbm.at[pl.ds(off, K)], gb[b], isg.at[b])
        pltpu.async_copy(dst_hbm.at[pl.ds(off, K)], db[b], isd.at[b])

    def _group(g, c):
        e0 = base_e + g * (NBUF * K)
        gds = []
        for b in range(NBUF):
            pltpu.make_async_copy(
                gidx_hbm.at[pl.ds(e0 + b * K, K)], gb[b], isg.at[b]).wait()
            pltpu.make_async_copy(
                dst_hbm.at[pl.ds(e0 + b * K, K)], db[b], isd.at[b]).wait()
            gds.append(pltpu.async_copy(zplus_hbm.at[gb[b]], rb[b], gsm.at[b]))
        mask_hi = jnp.full((16,), -65536, jnp.int32)
        sds = []
        for b in range(NBUF):
            gds[b].wait()

            def _edge4(j4, cc, _b=b):
                for u in range(4):
                    j = j4 * 4 + u
                    for q in range(4):
                        sl = pl.ds(q * 16, 16)
                        sh = pl.ds(HW + q * 16, 16)
                        w = rb[_b][j, sl]
                        ob[_b][j, sl] = plsc.bitcast(w << 16, _f32)
                        ob[_b][j, sh] = plsc.bitcast(w & mask_hi, _f32)
                return cc

            lax.fori_loop(0, K // 4, _edge4, 0)
            sds.append(pltpu.async_copy(ob[b], agg.at[db[b]], ssm.at[b], add=True))
        for b in range(NBUF):
            sds[b].wait()

        @pl.when(g < G_ITER - 1)
        def _():
            e1 = e0 + NBUF * K
            for b in range(NBUF):
                pltpu.async_copy(gidx_hbm.at[pl.ds(e1 + b * K, K)], gb[b], isg.at[b])
                pltpu.async_copy(dst_hbm.at[pl.ds(e1 + b * K, K)], db[b], isd.at[b])

        return c

    lax.fori_loop(0, G_ITER, _group, 0)
    plsc.subcore_barrier()

    pltpu.sync_copy(agg.at[pl.ds(row0, ROWS_PT), :],
                    out_hbm.at[cid, pl.ds(row0, ROWS_PT), :])


@functools.lru_cache(maxsize=1)
def _get_sc_agg():
    return functools.partial(
        pl.kernel,
        out_type=jax.ShapeDtypeStruct((NC, AGG_ROWS, H), _f32),
        compiler_params=pltpu.CompilerParams(use_tc_tiling_on_sc=False,
                                             needs_layout_passes=False),
        mesh=plsc.VectorSubcoreMesh(core_axis_name="c", subcore_axis_name="s",
                                    num_cores=NC, num_subcores=NS),
        scratch_types=(
            [pltpu.VMEM((16, H), _f32)]
            + [pltpu.VMEM((K,), jnp.int32) for _ in range(NBUF)]
            + [pltpu.VMEM((K,), jnp.int32) for _ in range(NBUF)]
            + [pltpu.VMEM((K, HW), jnp.int32) for _ in range(NBUF)]
            + [pltpu.VMEM((K, H), _f32) for _ in range(NBUF)]
            + [pltpu.VMEM_SHARED((AGG_ROWS, H), _f32)]
            + [pltpu.SemaphoreType.DMA((NBUF,)) for _ in range(4)]
        ),
    )(_sc_body)


def _mlp_body(last, z_ref, agg_ref, w1_ref, b1_ref, w2_ref, b2_ref, t_ref,
              batch_ref, zout_ref, pool_ref, zp_ref=None):
    b = pl.program_id(0)
    h = z_ref[...] + agg_ref[0] + agg_ref[1]
    h1 = jnp.maximum(
        jnp.dot(h, w1_ref[...], preferred_element_type=_f32) + b1_ref[...], 0.0)
    z2 = jnp.dot(h1, w2_ref[...], preferred_element_type=_f32) + b2_ref[...]
    if not last:
        z2 = jnp.maximum(z2, 0.0)
    zout_ref[...] = z2
    seg = batch_ref[0, 0, :]
    oh_t = (lax.broadcasted_iota(jnp.int32, (G, BLK), 0)
            == seg[None, :]).astype(_f32)
    contrib = jnp.dot(oh_t, z2, preferred_element_type=_f32)

    @pl.when(b == 0)
    def _():
        pool_ref[...] = jnp.zeros_like(pool_ref)

    pool_ref[...] += contrib
    if zp_ref is not None:
        for c in range(8):
            zp_ref[c] = _pack_rows(jnp.maximum(z2 + t_ref[c], 0.0))


def _make_mlp(last):
    out_specs = [
        pl.BlockSpec((BLK, H), lambda b: (b, 0)),
        pl.BlockSpec((G, H), lambda b: (0, 0)),
    ]
    out_shape = [
        jax.ShapeDtypeStruct((N, H), _f32),
        jax.ShapeDtypeStruct((G, H), _f32),
    ]
    if not last:
        out_specs.append(pl.BlockSpec((8, BLK, HW), lambda b: (0, b, 0)))
        out_shape.append(jax.ShapeDtypeStruct((8, N, HW), jnp.int32))
    return pl.pallas_call(
        functools.partial(_mlp_body, last),
        grid=(NBLK,),
        in_specs=[
            pl.BlockSpec((BLK, H), lambda b: (b, 0)),
            pl.BlockSpec((NC, BLK, H), lambda b: (0, b, 0)),
            pl.BlockSpec((H, 2 * H), lambda b: (0, 0)),
            pl.BlockSpec((1, 2 * H), lambda b: (0, 0)),
            pl.BlockSpec((2 * H, H), lambda b: (0, 0)),
            pl.BlockSpec((1, H), lambda b: (0, 0)),
            pl.BlockSpec((8, H), lambda b: (0, 0)),
            pl.BlockSpec((1, 1, BLK), lambda b: (b, 0, 0)),
        ],
        out_specs=out_specs,
        out_shape=out_shape,
    )


_mlp_mid = _make_mlp(False)
_mlp_last = _make_mlp(True)


def kernel(x, edge_index, edge_attr, batch, params):
    src = edge_index[0]
    dst = edge_index[1]
    srcp = jnp.concatenate(
        [src, jnp.arange(E_PAD - E, dtype=jnp.int32) % N])
    dstp = jnp.concatenate(
        [dst, N + jnp.arange(E_PAD - E, dtype=jnp.int32) % (AGG_ROWS - N)])
    attrp = jnp.pad(edge_attr.T, ((0, 0), (0, E_PAD - E)))
    gidx = _eidx(srcp.reshape(EROWS, 128),
                 attrp.reshape(3, EROWS, 128)).reshape(E_PAD)

    xf = jnp.pad(x.astype(_f32), ((0, 0), (0, 7)))
    d_mat = jnp.pad(
        jnp.stack([params['atom_emb'][i][1] - params['atom_emb'][i][0]
                   for i in range(9)]), ((0, 7), (0, 0)))
    base = sum(params['atom_emb'][i][0] for i in range(9))[None, :]
    b0, b1, b2 = params['bond_emb']
    i0 = jnp.array([0, 0, 0, 0, 1, 1, 1, 1], jnp.int32)
    i1 = jnp.array([0, 0, 1, 1, 0, 0, 1, 1], jnp.int32)
    i2 = jnp.array([0, 1, 0, 1, 0, 1, 0, 1], jnp.int32)
    t_tab = b0[i0] + b1[i1] + b2[i2]

    zcur, zpcur = _enc(xf, d_mat, base, t_tab)
    batch3 = batch.reshape(NBLK, 1, BLK)

    zs, gs = [], []
    for li, p in enumerate(params['layers']):
        s1 = BN_INV * p['g1']
        w1f = p['W1'] * s1[None, :]
        b1f = (p['b1'] * s1 + p['be1'])[None, :]
        s2 = BN_INV * p['g']
        w2f = p['W2'] * s2[None, :]
        b2f = (p['b2'] * s2 + p['be'])[None, :]
        agg = _get_sc_agg()(gidx, dstp, zpcur.reshape(8 * N, HW))
        if li < 2:
            zcur, pool, zpcur = _mlp_mid(zcur, agg, w1f, b1f, w2f, b2f,
                                         t_tab, batch3)
        else:
            zcur, pool = _mlp_last(zcur, agg, w1f, b1f, w2f, b2f,
                                   t_tab, batch3)
        zs.append(zcur)
        gs.append(pool)
    return jnp.concatenate(zs, axis=1), jnp.concatenate(gs, axis=1)
